# baseline (speedup 1.0000x reference)
import functools

import jax
import jax.numpy as jnp
from jax import lax
from jax.experimental import pallas as pl
from jax.experimental.pallas import tpu as pltpu
from jax.experimental.pallas import tpu_sc as plsc


def kernel(x, embedding):
    B, H = x.shape
    V, D = embedding.shape
    N = B * H

    info = plsc.get_sparse_core_info()
    NC, NS = info.num_cores, info.num_subcores
    NW = NC * NS

    K = 128
    n_chunks = N // (NW * K)
    assert N == NW * n_chunks * K

    xf = x.reshape(NW, n_chunks, K).astype(jnp.int32)

    mesh = plsc.VectorSubcoreMesh(core_axis_name="c", subcore_axis_name="s")

    @functools.partial(
        pl.kernel,
        out_type=jax.ShapeDtypeStruct((N, D), jnp.float32),
        mesh=mesh,
        scratch_types=[
            pltpu.VMEM((n_chunks, K), jnp.int32),
            pltpu.VMEM((K, D), jnp.float32),
            pltpu.SemaphoreType.DMA,
        ],
        compiler_params=pltpu.CompilerParams(use_tc_tiling_on_sc=False),
    )
    def emb_kernel(idx_hbm, table_hbm, out_hbm, idx_v, rows_v, sem):
        wid = lax.axis_index("s") * NC + lax.axis_index("c")
        base = wid * (n_chunks * K)
        pltpu.sync_copy(idx_hbm.at[wid], idx_v)

        def body(c, carry):
            pltpu.async_copy(table_hbm.at[idx_v.at[c]], rows_v, sem).wait()
            pltpu.sync_copy(rows_v, out_hbm.at[pl.ds(base + c * K, K)])
            return carry

        lax.fori_loop(0, n_chunks, body, 0)

    out = emb_kernel(xf, embedding)
    return out.reshape(B, H, D)


# speedup vs baseline: 1.1147x; 1.1147x over previous
import functools

import jax
import jax.numpy as jnp
from jax import lax
from jax.experimental import pallas as pl
from jax.experimental.pallas import tpu as pltpu
from jax.experimental.pallas import tpu_sc as plsc


def kernel(x, embedding):
    B, H = x.shape
    V, D = embedding.shape
    N = B * H

    info = plsc.get_sparse_core_info()
    NC, NS = info.num_cores, info.num_subcores
    NW = NC * NS

    K = 128
    NBUF = 8
    G = 4
    n_chunks = N // (NW * K)
    assert N == NW * n_chunks * K and n_chunks >= G
    SBYTES = K * D * 4

    xf = x.reshape(NW, n_chunks, K).astype(jnp.int32)

    mesh = plsc.VectorSubcoreMesh(core_axis_name="c", subcore_axis_name="s")

    @functools.partial(
        pl.kernel,
        out_type=jax.ShapeDtypeStruct((N, D), jnp.float32),
        mesh=mesh,
        scratch_types=[
            pltpu.VMEM((n_chunks, K), jnp.int32),
            pltpu.VMEM((NBUF, K, D), jnp.float32),
            pltpu.SemaphoreType.DMA((NBUF,)),
            pltpu.SemaphoreType.DMA((NBUF,)),
        ],
        compiler_params=pltpu.CompilerParams(use_tc_tiling_on_sc=False),
    )
    def emb_kernel(idx_hbm, table_hbm, out_hbm, idx_v, rows_v, gsem, ssem):
        wid = lax.axis_index("s") * NC + lax.axis_index("c")
        base = wid * (n_chunks * K)
        pltpu.sync_copy(idx_hbm.at[wid], idx_v)

        def store_chunk(cs, bs):
            pltpu.make_async_copy(
                table_hbm.at[idx_v.at[cs]], rows_v.at[bs], gsem.at[bs]
            ).wait()
            pltpu.async_copy(
                rows_v.at[bs], out_hbm.at[pl.ds(base + cs * K, K)], ssem.at[bs]
            )

        def body(c, carry):
            b = jnp.bitwise_and(c, NBUF - 1)

            @pl.when(c >= NBUF)
            def _():
                pltpu.make_async_copy(
                    rows_v.at[b], out_hbm.at[pl.ds(base, K)], ssem.at[b]
                ).wait()

            pltpu.async_copy(table_hbm.at[idx_v.at[c]], rows_v.at[b], gsem.at[b])

            @pl.when(c >= G)
            def _():
                cs = c - G
                store_chunk(cs, jnp.bitwise_and(cs, NBUF - 1))

            return carry

        lax.fori_loop(0, n_chunks, body, 0)

        for j in range(G):
            cs = n_chunks - G + j
            store_chunk(cs, cs % NBUF)
        for b in range(NBUF):
            pltpu.make_async_copy(
                rows_v.at[b], out_hbm.at[pl.ds(base, K)], ssem.at[b]
            ).wait()

    out = emb_kernel(xf, embedding)
    return out.reshape(B, H, D)
